# baseline (speedup 1.0000x reference)
import dataclasses

import numpy as np
import jax
import jax.numpy as jnp
from jax.experimental import pallas as pl
from jax.experimental.pallas import tpu as pltpu
from jax.experimental.pallas import tpu_sc as plsc

F = 26
FD = 4000
D = 16
B = 16384
NPAIR = F * (F - 1) // 2
KPAD = 768
NCHUNK = KPAD // 128
NGROUPS = 21
GPAD = NGROUPS * 16
INPUT_DIM = F * D + NPAIR
ROWW = F * D

NC, NS = 2, 16
NW = NC * NS
SPW = B // NW
G = 4
NIT = SPW // G

_foff = np.zeros(32, np.int32)
_foff[:F] = np.arange(F, dtype=np.int32) * FD
NGIDX = G * F
_sdx = np.zeros(112, np.int32)
_pf = np.zeros(112, np.int32)
_sdx[:NGIDX] = np.arange(NGIDX, dtype=np.int32) // F
_pf[:NGIDX] = np.arange(NGIDX, dtype=np.int32) % F
_lrow = np.zeros(GPAD, np.int32)
_lcol = np.zeros(GPAD, np.int32)
_rrow = np.zeros(GPAD, np.int32)
_rcol = np.zeros(GPAD, np.int32)
_p = 0
for _i in range(F - 1):
    for _j in range(_i + 1, F):
        _lrow[_p] = _i
        _lcol[_p] = D * _j
        _rrow[_p] = _j
        _rcol[_p] = D * (_i + 1)
        _p += 1
assert _p == NPAIR


def _sc_body(x_hbm, sdx_hbm, pf_hbm, lrow_hbm, lcol_hbm, rrow_hbm, rcol_hbm,
             emb_hbm, out_hbm,
             xq0, xq1, idx0, idx1, rows0, rows1, out0, out1,
             sdxv, pfv, lrowv, lcolv, rrowv, rcolv,
             gsem0, gsem1, osem0, osem1):
    wid = jax.lax.axis_index("s") * NC + jax.lax.axis_index("c")
    base = wid * SPW
    bufs = ((xq0, idx0, rows0, gsem0, out0, osem0),
            (xq1, idx1, rows1, gsem1, out1, osem1))

    pltpu.sync_copy(sdx_hbm, sdxv)
    pltpu.sync_copy(pf_hbm, pfv)
    pltpu.sync_copy(lrow_hbm, lrowv)
    pltpu.sync_copy(lcol_hbm, lcolv)
    pltpu.sync_copy(rrow_hbm, rrowv)
    pltpu.sync_copy(rcol_hbm, rcolv)
    zeros16 = jnp.zeros((16,), jnp.float32)
    for bf in bufs:
        for sb in range(G):
            bf[4][sb, pl.ds(INPUT_DIM + 11, 16)] = zeros16

    def fire(i, xq, idxv, rows, gsem):
        s0 = base + i * G
        pltpu.sync_copy(x_hbm.at[pl.ds(s0, G)], xq)

        @pl.loop(0, 7)
        def _(kk):
            sl = pl.ds(kk * 16, 16)
            sd = sdxv[sl]
            pf = pfv[sl]
            xg = plsc.load_gather(xq, [sd, pf])
            idxv[sl] = xg + pf * FD

        pltpu.async_copy(emb_hbm.at[idxv.at[pl.ds(0, NGIDX)]], rows, gsem)

    def drain_gathers(idxv, rows, gsem):
        pltpu.make_async_copy(emb_hbm.at[idxv.at[pl.ds(0, NGIDX)]], rows,
                              gsem).wait()

    def fire_out(i, outv, osem):
        s0 = base + i * G
        pltpu.async_copy(outv, out_hbm.at[pl.ds(s0, G)], osem)

    def drain_out(i, outv, osem):
        s0 = base + i * G
        pltpu.make_async_copy(outv, out_hbm.at[pl.ds(s0, G)], osem).wait()

    def compute(i, rows, outv, osem):
        for sb in range(G):
            sbase = sb * F

            @pl.loop(0, F)
            def _(cc):
                outv[sb, pl.ds(cc * D, D)] = rows[sbase + cc, pl.ds(0, D)]

            @pl.loop(0, NGROUPS)
            def _(g):
                sl = pl.ds(g * 16, 16)
                lr = lrowv[sl] + sbase
                lc = lcolv[sl]
                rr = rrowv[sl] + sbase
                rc = rcolv[sl]
                acc = jnp.zeros((16,), jnp.float32)
                for d in range(D):
                    a = plsc.load_gather(rows, [lr, lc + d])
                    b = plsc.load_gather(rows, [rr, rc + d])
                    acc = acc + a * b
                outv[sb, pl.ds(F * D + g * 16, 16)] = acc
        fire_out(i, outv, osem)

    fire(0, bufs[0][0], bufs[0][1], bufs[0][2], bufs[0][3])

    @pl.loop(0, NIT, step=2)
    def _(it):
        for p in range(2):
            i = it + p
            xq_n, idx_n, rows_n, gsem_n = bufs[1 - p][:4]
            xq_c, idx_c, rows_c, gsem_c, out_c, osem_c = bufs[p]

            @pl.when(i + 1 < NIT)
            def _():
                fire(i + 1, xq_n, idx_n, rows_n, gsem_n)

            drain_gathers(idx_c, rows_c, gsem_c)

            @pl.when(i >= 2)
            def _():
                drain_out(i, out_c, osem_c)

            compute(i, rows_c, out_c, osem_c)

    drain_out(NIT - 2, bufs[0][4], bufs[0][5])
    drain_out(NIT - 1, bufs[1][4], bufs[1][5])


def _sc_gather_ffm(xpad, embt):
    mesh = plsc.VectorSubcoreMesh(
        core_axis_name="c", subcore_axis_name="s", num_cores=NC,
        num_subcores=NS)
    cp = pltpu.CompilerParams(use_tc_tiling_on_sc=False)
    if "needs_layout_passes" in pltpu.CompilerParams.__dataclass_fields__:
        cp = dataclasses.replace(cp, needs_layout_passes=False)
    fn = pl.kernel(
        _sc_body,
        out_type=jax.ShapeDtypeStruct((B, KPAD), jnp.float32),
        mesh=mesh,
        compiler_params=cp,
        scratch_types=[
            pltpu.VMEM((G, 32), jnp.int32),
            pltpu.VMEM((G, 32), jnp.int32),
            pltpu.VMEM((112,), jnp.int32),
            pltpu.VMEM((112,), jnp.int32),
            pltpu.VMEM((NGIDX, ROWW), jnp.float32),
            pltpu.VMEM((NGIDX, ROWW), jnp.float32),
            pltpu.VMEM((G, KPAD), jnp.float32),
            pltpu.VMEM((G, KPAD), jnp.float32),
            pltpu.VMEM((112,), jnp.int32),
            pltpu.VMEM((112,), jnp.int32),
            pltpu.VMEM((GPAD,), jnp.int32),
            pltpu.VMEM((GPAD,), jnp.int32),
            pltpu.VMEM((GPAD,), jnp.int32),
            pltpu.VMEM((GPAD,), jnp.int32),
            pltpu.SemaphoreType.DMA,
            pltpu.SemaphoreType.DMA,
            pltpu.SemaphoreType.DMA,
            pltpu.SemaphoreType.DMA,
        ],
    )
    return fn(xpad, jnp.asarray(_sdx), jnp.asarray(_pf), jnp.asarray(_lrow),
              jnp.asarray(_lcol), jnp.asarray(_rrow), jnp.asarray(_rcol),
              embt)


TB = 2048
NT = B // TB


def _mlp_body(d0, d1, d2, d3, d4, d5, W1_ref, b1_ref, g1_ref, be1_ref,
              W2_ref, b2_ref, g2_ref, be2_ref, W3_ref, b3_ref, out_ref,
              h1_ref, h2_ref, s1_ref, q1_ref, s2_ref, q2_ref):
    p = pl.program_id(0)
    t = pl.program_id(1)
    inv_b = jnp.float32(1.0 / B)
    dnn_refs = (d0, d1, d2, d3, d4, d5)

    @pl.when(p == 0)
    def _():
        @pl.when(t == 0)
        def _():
            s1_ref[...] = jnp.zeros_like(s1_ref)
            q1_ref[...] = jnp.zeros_like(q1_ref)
            s2_ref[...] = jnp.zeros_like(s2_ref)
            q2_ref[...] = jnp.zeros_like(q2_ref)

        h1 = b1_ref[...]
        for c in range(NCHUNK):
            h1 = h1 + jnp.dot(dnn_refs[c][...],
                              W1_ref[pl.ds(c * 128, 128), :],
                              preferred_element_type=jnp.float32)
        h1_ref[pl.ds(t * TB, TB), :] = h1
        s1_ref[...] += jnp.sum(h1, axis=0, keepdims=True)
        q1_ref[...] += jnp.sum(h1 * h1, axis=0, keepdims=True)

    @pl.when(p == 1)
    def _():
        m1 = s1_ref[...] * inv_b
        v1 = q1_ref[...] * inv_b - m1 * m1
        inv1 = jax.lax.rsqrt(v1 + 1e-5)
        h1 = h1_ref[pl.ds(t * TB, TB), :]
        a1 = jnp.maximum((h1 - m1) * inv1 * g1_ref[...] + be1_ref[...], 0.0)
        h2 = jnp.dot(a1, W2_ref[...],
                     preferred_element_type=jnp.float32) + b2_ref[...]
        h2_ref[pl.ds(t * TB, TB), :] = h2
        s2_ref[...] += jnp.sum(h2, axis=0, keepdims=True)
        q2_ref[...] += jnp.sum(h2 * h2, axis=0, keepdims=True)

    @pl.when(p == 2)
    def _():
        m2 = s2_ref[...] * inv_b
        v2 = q2_ref[...] * inv_b - m2 * m2
        inv2 = jax.lax.rsqrt(v2 + 1e-5)
        h2 = h2_ref[pl.ds(t * TB, TB), :]
        a2 = jnp.maximum((h2 - m2) * inv2 * g2_ref[...] + be2_ref[...], 0.0)
        y = jnp.sum(a2 * W3_ref[...], axis=1, keepdims=True) + b3_ref[...]
        out_ref[...] = jax.nn.sigmoid(y)


def _tc_mlp(dnn6, W1p, b1, g1, be1, W2, b2, g2, be2, W3, b3):
    row = lambda v: v.reshape(1, -1)
    grid = (3, NT)

    def dnn_spec(c):
        return pl.BlockSpec((TB, 128),
                            lambda p, t, c=c: (jnp.where(p == 0, t, 0), c))

    out = pl.pallas_call(
        _mlp_body,
        grid=grid,
        in_specs=[dnn_spec(c) for c in range(NCHUNK)] + [
            pl.BlockSpec((KPAD, 64), lambda p, t: (0, 0)),
            pl.BlockSpec((1, 64), lambda p, t: (0, 0)),
            pl.BlockSpec((1, 64), lambda p, t: (0, 0)),
            pl.BlockSpec((1, 64), lambda p, t: (0, 0)),
            pl.BlockSpec((64, 32), lambda p, t: (0, 0)),
            pl.BlockSpec((1, 32), lambda p, t: (0, 0)),
            pl.BlockSpec((1, 32), lambda p, t: (0, 0)),
            pl.BlockSpec((1, 32), lambda p, t: (0, 0)),
            pl.BlockSpec((1, 32), lambda p, t: (0, 0)),
            pl.BlockSpec((1, 1), lambda p, t: (0, 0)),
        ],
        out_specs=pl.BlockSpec((TB, 1), lambda p, t: (t, 0)),
        out_shape=jax.ShapeDtypeStruct((B, 1), jnp.float32),
        scratch_shapes=[
            pltpu.VMEM((B, 64), jnp.float32),
            pltpu.VMEM((B, 32), jnp.float32),
            pltpu.VMEM((1, 64), jnp.float32),
            pltpu.VMEM((1, 64), jnp.float32),
            pltpu.VMEM((1, 32), jnp.float32),
            pltpu.VMEM((1, 32), jnp.float32),
        ],
    )(*([dnn6] * NCHUNK), W1p, row(b1), row(g1), row(be1), W2, row(b2),
      row(g2), row(be2), W3.reshape(1, -1), b3.reshape(1, 1))
    return out


def kernel(x, emb, W1, b1, g1, be1, W2, b2, g2, be2, W3, b3):
    embt = jnp.transpose(emb, (1, 0, 2)).reshape(FD * F, ROWW)
    xpad = jnp.pad(x, ((0, 0), (0, 32 - F)))
    dnn6 = _sc_gather_ffm(xpad, embt)
    W1p = jnp.concatenate(
        [W1, jnp.zeros((KPAD - INPUT_DIM, 64), jnp.float32)], axis=0)
    y = _tc_mlp(dnn6, W1p, b1, g1, be1, W2, b2, g2, be2, W3, b3)
    return jnp.squeeze(y, axis=1)


# speedup vs baseline: 1.6425x; 1.6425x over previous
import dataclasses

import numpy as np
import jax
import jax.numpy as jnp
from jax.experimental import pallas as pl
from jax.experimental.pallas import tpu as pltpu
from jax.experimental.pallas import tpu_sc as plsc

F = 26
FD = 4000
D = 16
B = 16384
NPAIR = F * (F - 1) // 2
KPAD = 768
NCHUNK = KPAD // 128
NGROUPS = 21
GPAD = NGROUPS * 16
INPUT_DIM = F * D + NPAIR
ROWW = F * D

NC, NS = 2, 16
NW = NC * NS
SPW = B // NW
G = 4
NIT = SPW // G

NROWS = 688
NLEFT = 26
NRIGHT = 351
_CHUNKS = [(c * 128, 128) for c in range(5)] + [(640, 48)]
_fsel = np.zeros(NROWS, np.int32)
_cbase = np.zeros(NROWS, np.int32)
for _f in range(F):
    _fsel[_f] = _f
    _cbase[_f] = FD * _f
_p = 0
for _i in range(F - 1):
    for _j in range(_i + 1, F):
        _fsel[NLEFT + _p] = _i
        _cbase[NLEFT + _p] = _j * F * FD + FD * _i
        _fsel[NRIGHT + _p] = _j
        _cbase[NRIGHT + _p] = (_i + 1) * F * FD + FD * _j
        _p += 1
assert _p == NPAIR


def _sc_body(x_hbm, fsel_hbm, cbase_hbm, emb_hbm, out_hbm,
             xq0, xq1, idx0, idx1, rows0, rows1, out0, out1,
             fselv, cbasev,
             gsem0, gsem1, osem0, osem1):
    wid = jax.lax.axis_index("s") * NC + jax.lax.axis_index("c")
    base = wid * SPW
    lane = jax.lax.iota(jnp.int32, 16)
    bufs = ((xq0, idx0, rows0, gsem0, out0, osem0),
            (xq1, idx1, rows1, gsem1, out1, osem1))

    pltpu.sync_copy(fsel_hbm, fselv)
    pltpu.sync_copy(cbase_hbm, cbasev)
    zeros16 = jnp.zeros((16,), jnp.float32)
    for bf in bufs:
        for sb in range(G):
            bf[4][sb, pl.ds(INPUT_DIM + 11, 16)] = zeros16

    def fire(i, xq, idxv, rows, gsem):
        s0 = base + i * G
        pltpu.sync_copy(x_hbm.at[pl.ds(s0, G)], xq)
        for sb in range(G):
            sbv = jnp.full((16,), sb, jnp.int32)

            @pl.loop(0, NROWS // 16)
            def _(kk):
                sl = pl.ds(kk * 16, 16)
                fv = fselv[sl]
                cb = cbasev[sl]
                xv = plsc.load_gather(xq, [sbv, fv])
                idxv[sb, sl] = xv + cb

        for sb in range(G):
            for off, ln in _CHUNKS:
                pltpu.async_copy(
                    emb_hbm.at[idxv.at[sb, pl.ds(off, ln)]],
                    rows.at[sb, pl.ds(off, ln)], gsem)

    def drain_gathers(idxv, rows, gsem):
        for sb in range(G):
            for off, ln in _CHUNKS:
                pltpu.make_async_copy(
                    emb_hbm.at[idxv.at[sb, pl.ds(off, ln)]],
                    rows.at[sb, pl.ds(off, ln)], gsem).wait()

    def fire_out(i, outv, osem):
        s0 = base + i * G
        pltpu.async_copy(outv, out_hbm.at[pl.ds(s0, G)], osem)

    def drain_out(i, outv, osem):
        s0 = base + i * G
        pltpu.make_async_copy(outv, out_hbm.at[pl.ds(s0, G)], osem).wait()

    def compute(i, rows, outv, osem):
        for sb in range(G):
            sbv = jnp.full((16,), sb, jnp.int32)

            @pl.loop(0, F)
            def _(cc):
                outv[sb, pl.ds(cc * D, D)] = rows[sb, cc, pl.ds(0, D)]

            @pl.loop(0, NGROUPS)
            def _(g):
                ra = NLEFT + g * 16 + lane
                rb = NRIGHT + g * 16 + lane
                acc = jnp.zeros((16,), jnp.float32)
                col = lane
                for d in range(D):
                    a = plsc.load_gather(rows, [sbv, ra, col])
                    b = plsc.load_gather(rows, [sbv, rb, col])
                    acc = acc + a * b
                    if d != D - 1:
                        col = (col + 1) & (D - 1)
                outv[sb, pl.ds(F * D + g * 16, 16)] = acc
        fire_out(i, outv, osem)

    fire(0, bufs[0][0], bufs[0][1], bufs[0][2], bufs[0][3])

    @pl.loop(0, NIT, step=2)
    def _(it):
        for p in range(2):
            i = it + p
            xq_n, idx_n, rows_n, gsem_n = bufs[1 - p][:4]
            xq_c, idx_c, rows_c, gsem_c, out_c, osem_c = bufs[p]

            @pl.when(i + 1 < NIT)
            def _():
                fire(i + 1, xq_n, idx_n, rows_n, gsem_n)

            drain_gathers(idx_c, rows_c, gsem_c)

            @pl.when(i >= 2)
            def _():
                drain_out(i, out_c, osem_c)

            compute(i, rows_c, out_c, osem_c)

    drain_out(NIT - 2, bufs[0][4], bufs[0][5])
    drain_out(NIT - 1, bufs[1][4], bufs[1][5])


def _sc_gather_ffm(xpad, embt):
    mesh = plsc.VectorSubcoreMesh(
        core_axis_name="c", subcore_axis_name="s", num_cores=NC,
        num_subcores=NS)
    cp = pltpu.CompilerParams(use_tc_tiling_on_sc=False)
    if "needs_layout_passes" in pltpu.CompilerParams.__dataclass_fields__:
        cp = dataclasses.replace(cp, needs_layout_passes=False)
    fn = pl.kernel(
        _sc_body,
        out_type=jax.ShapeDtypeStruct((B, KPAD), jnp.float32),
        mesh=mesh,
        compiler_params=cp,
        scratch_types=[
            pltpu.VMEM((G, 32), jnp.int32),
            pltpu.VMEM((G, 32), jnp.int32),
            pltpu.VMEM((G, NROWS), jnp.int32),
            pltpu.VMEM((G, NROWS), jnp.int32),
            pltpu.VMEM((G, NROWS, D), jnp.float32),
            pltpu.VMEM((G, NROWS, D), jnp.float32),
            pltpu.VMEM((G, KPAD), jnp.float32),
            pltpu.VMEM((G, KPAD), jnp.float32),
            pltpu.VMEM((NROWS,), jnp.int32),
            pltpu.VMEM((NROWS,), jnp.int32),
            pltpu.SemaphoreType.DMA,
            pltpu.SemaphoreType.DMA,
            pltpu.SemaphoreType.DMA,
            pltpu.SemaphoreType.DMA,
        ],
    )
    return fn(xpad, jnp.asarray(_fsel), jnp.asarray(_cbase), embt)


TB = 2048
NT = B // TB


def _mlp_body(d0, d1, d2, d3, d4, d5, W1_ref, b1_ref, g1_ref, be1_ref,
              W2_ref, b2_ref, g2_ref, be2_ref, W3_ref, b3_ref, out_ref,
              h1_ref, h2_ref, s1_ref, q1_ref, s2_ref, q2_ref):
    p = pl.program_id(0)
    t = pl.program_id(1)
    inv_b = jnp.float32(1.0 / B)
    dnn_refs = (d0, d1, d2, d3, d4, d5)

    @pl.when(p == 0)
    def _():
        @pl.when(t == 0)
        def _():
            s1_ref[...] = jnp.zeros_like(s1_ref)
            q1_ref[...] = jnp.zeros_like(q1_ref)
            s2_ref[...] = jnp.zeros_like(s2_ref)
            q2_ref[...] = jnp.zeros_like(q2_ref)

        h1 = b1_ref[...]
        for c in range(NCHUNK):
            h1 = h1 + jnp.dot(dnn_refs[c][...],
                              W1_ref[pl.ds(c * 128, 128), :],
                              preferred_element_type=jnp.float32)
        h1_ref[pl.ds(t * TB, TB), :] = h1
        s1_ref[...] += jnp.sum(h1, axis=0, keepdims=True)
        q1_ref[...] += jnp.sum(h1 * h1, axis=0, keepdims=True)

    @pl.when(p == 1)
    def _():
        m1 = s1_ref[...] * inv_b
        v1 = q1_ref[...] * inv_b - m1 * m1
        inv1 = jax.lax.rsqrt(v1 + 1e-5)
        h1 = h1_ref[pl.ds(t * TB, TB), :]
        a1 = jnp.maximum((h1 - m1) * inv1 * g1_ref[...] + be1_ref[...], 0.0)
        h2 = jnp.dot(a1, W2_ref[...],
                     preferred_element_type=jnp.float32) + b2_ref[...]
        h2_ref[pl.ds(t * TB, TB), :] = h2
        s2_ref[...] += jnp.sum(h2, axis=0, keepdims=True)
        q2_ref[...] += jnp.sum(h2 * h2, axis=0, keepdims=True)

    @pl.when(p == 2)
    def _():
        m2 = s2_ref[...] * inv_b
        v2 = q2_ref[...] * inv_b - m2 * m2
        inv2 = jax.lax.rsqrt(v2 + 1e-5)
        h2 = h2_ref[pl.ds(t * TB, TB), :]
        a2 = jnp.maximum((h2 - m2) * inv2 * g2_ref[...] + be2_ref[...], 0.0)
        y = jnp.sum(a2 * W3_ref[...], axis=1, keepdims=True) + b3_ref[...]
        out_ref[...] = jax.nn.sigmoid(y)


def _tc_mlp(dnn6, W1p, b1, g1, be1, W2, b2, g2, be2, W3, b3):
    row = lambda v: v.reshape(1, -1)
    grid = (3, NT)

    def dnn_spec(c):
        return pl.BlockSpec((TB, 128),
                            lambda p, t, c=c: (jnp.where(p == 0, t, 0), c))

    out = pl.pallas_call(
        _mlp_body,
        grid=grid,
        in_specs=[dnn_spec(c) for c in range(NCHUNK)] + [
            pl.BlockSpec((KPAD, 64), lambda p, t: (0, 0)),
            pl.BlockSpec((1, 64), lambda p, t: (0, 0)),
            pl.BlockSpec((1, 64), lambda p, t: (0, 0)),
            pl.BlockSpec((1, 64), lambda p, t: (0, 0)),
            pl.BlockSpec((64, 32), lambda p, t: (0, 0)),
            pl.BlockSpec((1, 32), lambda p, t: (0, 0)),
            pl.BlockSpec((1, 32), lambda p, t: (0, 0)),
            pl.BlockSpec((1, 32), lambda p, t: (0, 0)),
            pl.BlockSpec((1, 32), lambda p, t: (0, 0)),
            pl.BlockSpec((1, 1), lambda p, t: (0, 0)),
        ],
        out_specs=pl.BlockSpec((TB, 1), lambda p, t: (t, 0)),
        out_shape=jax.ShapeDtypeStruct((B, 1), jnp.float32),
        scratch_shapes=[
            pltpu.VMEM((B, 64), jnp.float32),
            pltpu.VMEM((B, 32), jnp.float32),
            pltpu.VMEM((1, 64), jnp.float32),
            pltpu.VMEM((1, 64), jnp.float32),
            pltpu.VMEM((1, 32), jnp.float32),
            pltpu.VMEM((1, 32), jnp.float32),
        ],
    )(*([dnn6] * NCHUNK), W1p, row(b1), row(g1), row(be1), W2, row(b2),
      row(g2), row(be2), W3.reshape(1, -1), b3.reshape(1, 1))
    return out


def kernel(x, emb, W1, b1, g1, be1, W2, b2, g2, be2, W3, b3):
    embt = emb.reshape(F * F * FD, D)
    xpad = jnp.pad(x, ((0, 0), (0, 32 - F)))
    dnn6 = _sc_gather_ffm(xpad, embt)
    W1p = jnp.concatenate(
        [W1, jnp.zeros((KPAD - INPUT_DIM, 64), jnp.float32)], axis=0)
    y = _tc_mlp(dnn6, W1p, b1, g1, be1, W2, b2, g2, be2, W3, b3)
    return jnp.squeeze(y, axis=1)


# speedup vs baseline: 1.6788x; 1.0221x over previous
import dataclasses

import numpy as np
import jax
import jax.numpy as jnp
from jax.experimental import pallas as pl
from jax.experimental.pallas import tpu as pltpu
from jax.experimental.pallas import tpu_sc as plsc

F = 26
FD = 4000
D = 16
B = 16384
NPAIR = F * (F - 1) // 2
KPAD = 768
NCHUNK = KPAD // 128
NGROUPS = 21
GPAD = NGROUPS * 16
INPUT_DIM = F * D + NPAIR
ROWW = F * D

NC, NS = 2, 16
NW = NC * NS
SPW = B // NW
G = 4
NIT = SPW // G

NROWS = 688
NLEFT = 26
NRIGHT = 351
_CHUNKS = [(c * 128, 128) for c in range(5)] + [(640, 48)]
_fsel = np.zeros(NROWS, np.int32)
_cbase = np.zeros(NROWS, np.int32)
for _f in range(F):
    _fsel[_f] = _f
    _cbase[_f] = FD * _f
_p = 0
for _i in range(F - 1):
    for _j in range(_i + 1, F):
        _fsel[NLEFT + _p] = _i
        _cbase[NLEFT + _p] = _j * F * FD + FD * _i
        _fsel[NRIGHT + _p] = _j
        _cbase[NRIGHT + _p] = (_i + 1) * F * FD + FD * _j
        _p += 1
assert _p == NPAIR


def _sc_body(x_hbm, fsel_hbm, cbase_hbm, emb_hbm, out_hbm,
             xq0, xq1, idx0, idx1, rows0, rows1, out0, out1,
             fselv, cbasev,
             gsem0, gsem1, osem0, osem1):
    wid = jax.lax.axis_index("s") * NC + jax.lax.axis_index("c")
    base = wid * SPW
    lane = jax.lax.iota(jnp.int32, 16)
    bufs = ((xq0, idx0, rows0, gsem0, out0, osem0),
            (xq1, idx1, rows1, gsem1, out1, osem1))

    pltpu.sync_copy(fsel_hbm, fselv)
    pltpu.sync_copy(cbase_hbm, cbasev)
    zeros16 = jnp.zeros((16,), jnp.float32)
    for bf in bufs:
        for sb in range(G):
            bf[4][sb, pl.ds(INPUT_DIM + 11, 16)] = zeros16

    def fire(i, xq, idxv, rows, gsem):
        s0 = base + i * G
        pltpu.sync_copy(x_hbm.at[pl.ds(s0, G)], xq)
        for sb in range(G):
            sbv = jnp.full((16,), sb, jnp.int32)

            @pl.loop(0, NROWS // 16)
            def _(kk):
                sl = pl.ds(kk * 16, 16)
                fv = fselv[sl]
                cb = cbasev[sl]
                xv = plsc.load_gather(xq, [sbv, fv])
                idxv[sb, sl] = xv + cb

        for sb in range(G):
            for off, ln in _CHUNKS:
                pltpu.async_copy(
                    emb_hbm.at[idxv.at[sb, pl.ds(off, ln)]],
                    rows.at[sb, pl.ds(off, ln)], gsem)

    def drain_gathers(idxv, rows, gsem):
        for sb in range(G):
            for off, ln in _CHUNKS:
                pltpu.make_async_copy(
                    emb_hbm.at[idxv.at[sb, pl.ds(off, ln)]],
                    rows.at[sb, pl.ds(off, ln)], gsem).wait()

    def fire_out(i, outv, osem):
        s0 = base + i * G
        for c in range(NCHUNK):
            pltpu.async_copy(outv.at[:, pl.ds(c * 128, 128)],
                             out_hbm.at[pl.ds(c * B + s0, G)], osem)

    def drain_out(i, outv, osem):
        s0 = base + i * G
        for c in range(NCHUNK):
            pltpu.make_async_copy(outv.at[:, pl.ds(c * 128, 128)],
                                  out_hbm.at[pl.ds(c * B + s0, G)],
                                  osem).wait()

    def compute(i, rows, outv, osem):
        for sb in range(G):
            sbv = jnp.full((16,), sb, jnp.int32)

            @pl.loop(0, F)
            def _(cc):
                outv[sb, pl.ds(cc * D, D)] = rows[sb, cc, pl.ds(0, D)]

            @pl.loop(0, NGROUPS)
            def _(g):
                ra = NLEFT + g * 16 + lane
                rb = NRIGHT + g * 16 + lane
                acc = jnp.zeros((16,), jnp.float32)
                col = lane
                for d in range(D):
                    a = plsc.load_gather(rows, [sbv, ra, col])
                    b = plsc.load_gather(rows, [sbv, rb, col])
                    acc = acc + a * b
                    if d != D - 1:
                        col = (col + 1) & (D - 1)
                outv[sb, pl.ds(F * D + g * 16, 16)] = acc
        fire_out(i, outv, osem)

    fire(0, bufs[0][0], bufs[0][1], bufs[0][2], bufs[0][3])

    @pl.loop(0, NIT, step=2)
    def _(it):
        for p in range(2):
            i = it + p
            xq_n, idx_n, rows_n, gsem_n = bufs[1 - p][:4]
            xq_c, idx_c, rows_c, gsem_c, out_c, osem_c = bufs[p]

            @pl.when(i + 1 < NIT)
            def _():
                fire(i + 1, xq_n, idx_n, rows_n, gsem_n)

            drain_gathers(idx_c, rows_c, gsem_c)

            @pl.when(i >= 2)
            def _():
                drain_out(i, out_c, osem_c)

            compute(i, rows_c, out_c, osem_c)

    drain_out(NIT - 2, bufs[0][4], bufs[0][5])
    drain_out(NIT - 1, bufs[1][4], bufs[1][5])


def _sc_gather_ffm(xpad, embt):
    mesh = plsc.VectorSubcoreMesh(
        core_axis_name="c", subcore_axis_name="s", num_cores=NC,
        num_subcores=NS)
    cp = pltpu.CompilerParams(use_tc_tiling_on_sc=False)
    if "needs_layout_passes" in pltpu.CompilerParams.__dataclass_fields__:
        cp = dataclasses.replace(cp, needs_layout_passes=False)
    fn = pl.kernel(
        _sc_body,
        out_type=jax.ShapeDtypeStruct((NCHUNK * B, 128), jnp.float32),
        mesh=mesh,
        compiler_params=cp,
        scratch_types=[
            pltpu.VMEM((G, 32), jnp.int32),
            pltpu.VMEM((G, 32), jnp.int32),
            pltpu.VMEM((G, NROWS), jnp.int32),
            pltpu.VMEM((G, NROWS), jnp.int32),
            pltpu.VMEM((G, NROWS, D), jnp.float32),
            pltpu.VMEM((G, NROWS, D), jnp.float32),
            pltpu.VMEM((G, KPAD), jnp.float32),
            pltpu.VMEM((G, KPAD), jnp.float32),
            pltpu.VMEM((NROWS,), jnp.int32),
            pltpu.VMEM((NROWS,), jnp.int32),
            pltpu.SemaphoreType.DMA,
            pltpu.SemaphoreType.DMA,
            pltpu.SemaphoreType.DMA,
            pltpu.SemaphoreType.DMA,
        ],
    )
    return fn(xpad, jnp.asarray(_fsel), jnp.asarray(_cbase), embt)


TB = 2048
NT = B // TB


def _mlp_body(d0, d1, d2, d3, d4, d5, W1_ref, b1_ref, g1_ref, be1_ref,
              W2_ref, b2_ref, g2_ref, be2_ref, W3_ref, b3_ref, out_ref,
              h1_ref, h2_ref, s1_ref, q1_ref, s2_ref, q2_ref):
    p = pl.program_id(0)
    t = pl.program_id(1)
    inv_b = jnp.float32(1.0 / B)
    dnn_refs = (d0, d1, d2, d3, d4, d5)

    @pl.when(p == 0)
    def _():
        @pl.when(t == 0)
        def _():
            s1_ref[...] = jnp.zeros_like(s1_ref)
            q1_ref[...] = jnp.zeros_like(q1_ref)
            s2_ref[...] = jnp.zeros_like(s2_ref)
            q2_ref[...] = jnp.zeros_like(q2_ref)

        h1 = b1_ref[...]
        for c in range(NCHUNK):
            h1 = h1 + jnp.dot(dnn_refs[c][...],
                              W1_ref[pl.ds(c * 128, 128), :],
                              preferred_element_type=jnp.float32)
        h1_ref[pl.ds(t * TB, TB), :] = h1
        s1_ref[...] += jnp.sum(h1, axis=0, keepdims=True)
        q1_ref[...] += jnp.sum(h1 * h1, axis=0, keepdims=True)

    @pl.when(p == 1)
    def _():
        m1 = s1_ref[...] * inv_b
        v1 = q1_ref[...] * inv_b - m1 * m1
        inv1 = jax.lax.rsqrt(v1 + 1e-5)
        h1 = h1_ref[pl.ds(t * TB, TB), :]
        a1 = jnp.maximum((h1 - m1) * inv1 * g1_ref[...] + be1_ref[...], 0.0)
        h2 = jnp.dot(a1, W2_ref[...],
                     preferred_element_type=jnp.float32) + b2_ref[...]
        h2_ref[pl.ds(t * TB, TB), :] = h2
        s2_ref[...] += jnp.sum(h2, axis=0, keepdims=True)
        q2_ref[...] += jnp.sum(h2 * h2, axis=0, keepdims=True)

    @pl.when(p == 2)
    def _():
        m2 = s2_ref[...] * inv_b
        v2 = q2_ref[...] * inv_b - m2 * m2
        inv2 = jax.lax.rsqrt(v2 + 1e-5)
        h2 = h2_ref[pl.ds(t * TB, TB), :]
        a2 = jnp.maximum((h2 - m2) * inv2 * g2_ref[...] + be2_ref[...], 0.0)
        y = jnp.sum(a2 * W3_ref[...], axis=1, keepdims=True) + b3_ref[...]
        out_ref[...] = jax.nn.sigmoid(y)


def _tc_mlp(dnn6, W1p, b1, g1, be1, W2, b2, g2, be2, W3, b3):
    row = lambda v: v.reshape(1, -1)
    grid = (3, NT)

    def dnn_spec(c):
        return pl.BlockSpec(
            (TB, 128),
            lambda p, t, c=c: (c * NT + jnp.where(p == 0, t, 0), 0))

    out = pl.pallas_call(
        _mlp_body,
        grid=grid,
        in_specs=[dnn_spec(c) for c in range(NCHUNK)] + [
            pl.BlockSpec((KPAD, 64), lambda p, t: (0, 0)),
            pl.BlockSpec((1, 64), lambda p, t: (0, 0)),
            pl.BlockSpec((1, 64), lambda p, t: (0, 0)),
            pl.BlockSpec((1, 64), lambda p, t: (0, 0)),
            pl.BlockSpec((64, 32), lambda p, t: (0, 0)),
            pl.BlockSpec((1, 32), lambda p, t: (0, 0)),
            pl.BlockSpec((1, 32), lambda p, t: (0, 0)),
            pl.BlockSpec((1, 32), lambda p, t: (0, 0)),
            pl.BlockSpec((1, 32), lambda p, t: (0, 0)),
            pl.BlockSpec((1, 1), lambda p, t: (0, 0)),
        ],
        out_specs=pl.BlockSpec((TB, 1), lambda p, t: (t, 0)),
        out_shape=jax.ShapeDtypeStruct((B, 1), jnp.float32),
        scratch_shapes=[
            pltpu.VMEM((B, 64), jnp.float32),
            pltpu.VMEM((B, 32), jnp.float32),
            pltpu.VMEM((1, 64), jnp.float32),
            pltpu.VMEM((1, 64), jnp.float32),
            pltpu.VMEM((1, 32), jnp.float32),
            pltpu.VMEM((1, 32), jnp.float32),
        ],
    )(*([dnn6] * NCHUNK), W1p, row(b1), row(g1), row(be1), W2, row(b2),
      row(g2), row(be2), W3.reshape(1, -1), b3.reshape(1, 1))
    return out


def kernel(x, emb, W1, b1, g1, be1, W2, b2, g2, be2, W3, b3):
    embt = emb.reshape(F * F * FD, D)
    xpad = jnp.pad(x, ((0, 0), (0, 32 - F)))
    dnn6 = _sc_gather_ffm(xpad, embt)
    W1p = jnp.concatenate(
        [W1, jnp.zeros((KPAD - INPUT_DIM, 64), jnp.float32)], axis=0)
    y = _tc_mlp(dnn6, W1p, b1, g1, be1, W2, b2, g2, be2, W3, b3)
    return jnp.squeeze(y, axis=1)
